# speedup vs baseline: 1.1162x; 1.1162x over previous
import functools

import jax
import jax.numpy as jnp
from jax.experimental import pallas as pl
from jax.experimental.pallas import tpu as pltpu

H = 1024
E = 8
NREG = 5
NSH = 2
NEX = NREG + NSH
P2 = 16
LBW = 0.01


def _router_body(x_ref, qf_ref, ln_g, ln_b, qw1, qb1, qw2, qb2, fgx, fgq, fgb,
                 wg1, wg2, cewg, cefw, cefb, cec, cpw, cpb,
                 xn_out, wd_out, part_out, disp_out, dens_out):
    f32 = jnp.float32
    x = x_ref[...]
    mu = jnp.mean(x, axis=-1, keepdims=True)
    xc = x - mu
    var = jnp.mean(xc * xc, axis=-1, keepdims=True)
    xn = xc / jnp.sqrt(var + 1e-5) * ln_g[...] + ln_b[...]

    q = jnp.maximum(
        jnp.dot(qf_ref[...], qw1[...], preferred_element_type=f32) + qb1[...], 0.0)
    q = jnp.dot(q, qw2[...], preferred_element_type=f32) + qb2[...]
    fused = jnp.maximum(
        jnp.dot(xn, fgx[...], preferred_element_type=f32)
        + jnp.dot(q, fgq[...], preferred_element_type=f32) + fgb[...], 0.0)
    tl = jnp.tanh(jnp.dot(fused, wg1[...], preferred_element_type=f32))
    logits = jnp.dot(tl, wg2[...], preferred_element_type=f32)

    m = jnp.max(logits, axis=-1, keepdims=True)
    ez = jnp.exp(logits - m)
    ew = ez / jnp.sum(ez, axis=-1, keepdims=True)

    iota = jax.lax.broadcasted_iota(jnp.int32, ew.shape, 1)
    m1 = jnp.max(ew, axis=-1, keepdims=True)
    i1 = jnp.min(jnp.where(ew == m1, iota, E), axis=-1, keepdims=True)
    ewm = jnp.where(iota == i1, -1.0, ew)
    m2 = jnp.max(ewm, axis=-1, keepdims=True)
    i2 = jnp.min(jnp.where(ewm == m2, iota, E), axis=-1, keepdims=True)
    s = m1 + m2 + 1e-6
    wd = jnp.where(iota == i1, m1 / s, 0.0) + jnp.where(iota == i2, m2 / s, 0.0)

    z = jnp.dot(xn, cewg[...], preferred_element_type=f32)
    zm = jnp.max(z, axis=-1, keepdims=True)
    zez = jnp.exp(z - zm)
    cw = zez / jnp.sum(zez, axis=-1, keepdims=True)
    fc = jnp.dot(xn, cefw[...], preferred_element_type=f32) + cefb[...]
    ce_out = cw[:, 0:1] * fc + cw[:, 1:2] * cec[...]
    cp_out = jnp.dot(xn, cpw[...], preferred_element_type=f32) + cpb[...]
    part = wd[:, 5:6] * ce_out + wd[:, 6:7] * cp_out

    xn_out[...] = xn.astype(jnp.bfloat16)
    wd_out[...] = wd
    part_out[...] = part
    disp_out[...] = jnp.sum((wd > 0.0).astype(f32), axis=0, keepdims=True).reshape(1, 1, E)
    dens_out[...] = jnp.sum(ew, axis=0, keepdims=True).reshape(1, 1, E)


def _expert_body(nt, nb, xn_ref, wd_ref, part_ref, mw1, mb1, mw2, mb2, mw3, mb3,
                 opw0, opw1, opw2, opb, disp_ref, dens_ref,
                 gauss_out, aux_out, acc_ref, sh_ref):
    f32 = jnp.float32
    t = pl.program_id(0)
    e = pl.program_id(1)
    x = xn_ref[...]
    h = jnp.maximum(jnp.dot(x, mw1[0], preferred_element_type=f32) + mb1[0], 0.0)
    h = jnp.maximum(
        jnp.dot(h.astype(jnp.bfloat16), mw2[0], preferred_element_type=f32) + mb2[0], 0.0)
    out = jnp.dot(h.astype(jnp.bfloat16), mw3[0], preferred_element_type=f32) + mb3[0]

    lane = jax.lax.broadcasted_iota(jnp.int32, (x.shape[0], E), 1)
    w_col = jnp.sum(jnp.where(lane == e, wd_ref[...], 0.0), axis=-1, keepdims=True)

    @pl.when(e == 0)
    def _():
        acc_ref[...] = part_ref[...] + w_col * out

    @pl.when(jnp.logical_and(e > 0, e < NREG))
    def _():
        acc_ref[...] = acc_ref[...] + w_col * out

    @pl.when(e == NREG)
    def _():
        sh_ref[...] = out

    @pl.when(e == NREG + 1)
    def _():
        g = (jnp.dot(acc_ref[...], opw0[...], preferred_element_type=f32)
             + jnp.dot(sh_ref[...], opw1[...], preferred_element_type=f32)
             + jnp.dot(out, opw2[...], preferred_element_type=f32) + opb[...])
        gauss_out[...] = g

    @pl.when(jnp.logical_and(t == nt - 1, e == NREG + 1))
    def _():
        cnt = jnp.sum(disp_ref[...], axis=0)
        dsum = jnp.sum(dens_ref[...], axis=0)
        val = (E * LBW) * jnp.sum(cnt * dsum) / (nb * nb)
        aux_out[...] = val.reshape(1, 1)


def kernel(multimodal_feat, query_feat, ln_g, ln_b, qe_w1, qe_b1, qe_w2, qe_b2,
           fg_w, fg_b, wg1, wg2, pw1, pb1, pw2, pb2, pw3, pb3, ce_const, ce_wg,
           ce_fc_w, ce_fc_b, cp_w, cp_b, sw1, sb1, sw2, sb2, sw3, sb3, op_w, op_b):
    f32 = jnp.float32
    bf = jnp.bfloat16
    nb = multimodal_feat.shape[0]
    bta = 1024
    btb = 1024
    na = nb // bta
    ntb = nb // btb

    r2 = lambda v: v.reshape(1, -1)
    full = lambda a: pl.BlockSpec(a.shape, lambda t, _n=None: (0,) * a.ndim)

    ins_a = [
        multimodal_feat, query_feat, r2(ln_g), r2(ln_b),
        qe_w1, r2(qe_b1), qe_w2, r2(qe_b2),
        fg_w[:H], fg_w[H:], r2(fg_b), wg1, wg2,
        ce_wg, ce_fc_w, r2(ce_fc_b), r2(ce_const), cp_w, r2(cp_b),
    ]
    specs_a = [
        pl.BlockSpec((bta, H), lambda t: (t, 0)),
        pl.BlockSpec((bta, H), lambda t: (t, 0)),
    ] + [pl.BlockSpec(a.shape, functools.partial(lambda nd, t: (0,) * nd, a.ndim))
         for a in ins_a[2:]]
    out_shape_a = [
        jax.ShapeDtypeStruct((nb, H), bf),
        jax.ShapeDtypeStruct((nb, E), f32),
        jax.ShapeDtypeStruct((nb, P2), f32),
        jax.ShapeDtypeStruct((na, 1, E), f32),
        jax.ShapeDtypeStruct((na, 1, E), f32),
    ]
    out_specs_a = [
        pl.BlockSpec((bta, H), lambda t: (t, 0)),
        pl.BlockSpec((bta, E), lambda t: (t, 0)),
        pl.BlockSpec((bta, P2), lambda t: (t, 0)),
        pl.BlockSpec((1, 1, E), lambda t: (t, 0, 0)),
        pl.BlockSpec((1, 1, E), lambda t: (t, 0, 0)),
    ]
    xn_bf, wd, part, disp, dens = pl.pallas_call(
        _router_body, grid=(na,), in_specs=specs_a,
        out_specs=out_specs_a, out_shape=out_shape_a,
    )(*ins_a)

    mw1 = jnp.concatenate([pw1, sw1], axis=0).astype(bf)
    mw2 = jnp.concatenate([pw2, sw2], axis=0).astype(bf)
    mw3 = jnp.concatenate([pw3, sw3], axis=0).astype(bf)
    mb1 = jnp.concatenate([pb1, sb1], axis=0).reshape(NEX, 1, 2 * H)
    mb2 = jnp.concatenate([pb2, sb2], axis=0).reshape(NEX, 1, H)
    mb3 = jnp.concatenate([pb3, sb3], axis=0).reshape(NEX, 1, P2)

    ins_b = [
        xn_bf, wd, part,
        mw1, mb1, mw2, mb2, mw3, mb3,
        op_w[0:P2], op_w[P2:2 * P2], op_w[2 * P2:], r2(op_b),
        disp, dens,
    ]
    specs_b = [
        pl.BlockSpec((btb, H), lambda t, e: (t, 0)),
        pl.BlockSpec((btb, E), lambda t, e: (t, 0)),
        pl.BlockSpec((btb, P2), lambda t, e: (t, 0)),
        pl.BlockSpec((1, H, 2 * H), lambda t, e: (e, 0, 0)),
        pl.BlockSpec((1, 1, 2 * H), lambda t, e: (e, 0, 0)),
        pl.BlockSpec((1, 2 * H, H), lambda t, e: (e, 0, 0)),
        pl.BlockSpec((1, 1, H), lambda t, e: (e, 0, 0)),
        pl.BlockSpec((1, H, P2), lambda t, e: (e, 0, 0)),
        pl.BlockSpec((1, 1, P2), lambda t, e: (e, 0, 0)),
        pl.BlockSpec((P2, P2), lambda t, e: (0, 0)),
        pl.BlockSpec((P2, P2), lambda t, e: (0, 0)),
        pl.BlockSpec((P2, P2), lambda t, e: (0, 0)),
        pl.BlockSpec((1, P2), lambda t, e: (0, 0)),
        pl.BlockSpec((na, 1, E), lambda t, e: (0, 0, 0)),
        pl.BlockSpec((na, 1, E), lambda t, e: (0, 0, 0)),
    ]
    out_shape_b = [
        jax.ShapeDtypeStruct((nb, P2), f32),
        jax.ShapeDtypeStruct((1, 1), f32),
    ]
    out_specs_b = [
        pl.BlockSpec((btb, P2), lambda t, e: (t, 0)),
        pl.BlockSpec((1, 1), lambda t, e: (0, 0)),
    ]
    gauss, aux = pl.pallas_call(
        functools.partial(_expert_body, ntb, float(nb)),
        grid=(ntb, NEX),
        in_specs=specs_b, out_specs=out_specs_b, out_shape=out_shape_b,
        scratch_shapes=[
            pltpu.VMEM((btb, P2), f32),
            pltpu.VMEM((btb, P2), f32),
        ],
    )(*ins_b)

    return gauss.reshape(-1, 2), aux[0, 0]


# speedup vs baseline: 1.2487x; 1.1187x over previous
import functools

import jax
import jax.numpy as jnp
from jax.experimental import pallas as pl
from jax.experimental.pallas import tpu as pltpu

H = 1024
E = 8
NREG = 5
NSH = 2
P2 = 16
LBW = 0.01


def _router_body(x_ref, qf_ref, ln_g, ln_b, qw1, qb1, qw2, qb2, fgx, fgq, fgb,
                 wg1, wg2, cewg, cefw, cefb, cec, cpw, cpb,
                 xn_out, wd_out, part_out, disp_out, dens_out):
    f32 = jnp.float32
    x = x_ref[...]
    mu = jnp.mean(x, axis=-1, keepdims=True)
    xc = x - mu
    var = jnp.mean(xc * xc, axis=-1, keepdims=True)
    xn = xc / jnp.sqrt(var + 1e-5) * ln_g[...] + ln_b[...]

    q = jnp.maximum(
        jnp.dot(qf_ref[...], qw1[...], preferred_element_type=f32) + qb1[...], 0.0)
    q = jnp.dot(q, qw2[...], preferred_element_type=f32) + qb2[...]
    fused = jnp.maximum(
        jnp.dot(xn, fgx[...], preferred_element_type=f32)
        + jnp.dot(q, fgq[...], preferred_element_type=f32) + fgb[...], 0.0)
    tl = jnp.tanh(jnp.dot(fused, wg1[...], preferred_element_type=f32))
    logits = jnp.dot(tl, wg2[...], preferred_element_type=f32)

    m = jnp.max(logits, axis=-1, keepdims=True)
    ez = jnp.exp(logits - m)
    ew = ez / jnp.sum(ez, axis=-1, keepdims=True)

    iota = jax.lax.broadcasted_iota(jnp.int32, ew.shape, 1)
    m1 = jnp.max(ew, axis=-1, keepdims=True)
    i1 = jnp.min(jnp.where(ew == m1, iota, E), axis=-1, keepdims=True)
    ewm = jnp.where(iota == i1, -1.0, ew)
    m2 = jnp.max(ewm, axis=-1, keepdims=True)
    i2 = jnp.min(jnp.where(ewm == m2, iota, E), axis=-1, keepdims=True)
    s = m1 + m2 + 1e-6
    wd = jnp.where(iota == i1, m1 / s, 0.0) + jnp.where(iota == i2, m2 / s, 0.0)

    z = jnp.dot(xn, cewg[...], preferred_element_type=f32)
    zm = jnp.max(z, axis=-1, keepdims=True)
    zez = jnp.exp(z - zm)
    cw = zez / jnp.sum(zez, axis=-1, keepdims=True)
    fc = jnp.dot(xn, cefw[...], preferred_element_type=f32) + cefb[...]
    ce_out = cw[:, 0:1] * fc + cw[:, 1:2] * cec[...]
    cp_out = jnp.dot(xn, cpw[...], preferred_element_type=f32) + cpb[...]
    part = wd[:, 5:6] * ce_out + wd[:, 6:7] * cp_out

    xn_out[...] = xn
    wd_out[...] = wd
    part_out[...] = part
    disp_out[...] = jnp.sum((wd > 0.0).astype(f32), axis=0, keepdims=True).reshape(1, 1, E)
    dens_out[...] = jnp.sum(ew, axis=0, keepdims=True).reshape(1, 1, E)


def _mlp(x, w1, b1, w2, b2, w3, b3):
    f32 = jnp.float32
    h = jnp.maximum(jnp.dot(x, w1, preferred_element_type=f32) + b1, 0.0)
    h = jnp.maximum(jnp.dot(h, w2, preferred_element_type=f32) + b2, 0.0)
    return jnp.dot(h, w3, preferred_element_type=f32) + b3


def _mlp_split(x, w1, b1, w2, b2, w3, b3):
    n = x.shape[0] // 2
    return jnp.concatenate(
        [_mlp(x[:n], w1, b1, w2, b2, w3, b3),
         _mlp(x[n:], w1, b1, w2, b2, w3, b3)], axis=0)


def _shared_body(xn_ref, w1a, b1a, w2a, b2a, w3a, b3a,
                 w1b, b1b, w2b, b2b, w3b, b3b, s0_out, s1_out):
    x = xn_ref[...]
    s0_out[...] = _mlp_split(x, w1a[...], b1a[...], w2a[...], b2a[...],
                             w3a[...], b3a[...])
    s1_out[...] = _mlp_split(x, w1b[...], b1b[...], w2b[...], b2b[...],
                             w3b[...], b3b[...])


def _routed_body(nt, nb, xn_ref, wd_ref, part_ref, s0_ref, s1_ref,
                 pw1_r, pb1_r, pw2_r, pb2_r, pw3_r, pb3_r,
                 opw0, opw1, opw2, opb, disp_ref, dens_ref,
                 gauss_out, aux_out, acc_ref):
    f32 = jnp.float32
    t = pl.program_id(0)
    e = pl.program_id(1)
    x = xn_ref[...]
    out = _mlp_split(x, pw1_r[0], pb1_r[0], pw2_r[0], pb2_r[0],
                     pw3_r[0], pb3_r[0])

    lane = jax.lax.broadcasted_iota(jnp.int32, (x.shape[0], E), 1)
    w_col = jnp.sum(jnp.where(lane == e, wd_ref[...], 0.0), axis=-1, keepdims=True)

    @pl.when(e == 0)
    def _():
        acc_ref[...] = part_ref[...] + w_col * out

    @pl.when(e > 0)
    def _():
        acc_ref[...] = acc_ref[...] + w_col * out

    @pl.when(e == NREG - 1)
    def _():
        g = (jnp.dot(acc_ref[...], opw0[...], preferred_element_type=f32)
             + jnp.dot(s0_ref[...], opw1[...], preferred_element_type=f32)
             + jnp.dot(s1_ref[...], opw2[...], preferred_element_type=f32)
             + opb[...])
        gauss_out[...] = g

    @pl.when(jnp.logical_and(t == nt - 1, e == NREG - 1))
    def _():
        cnt = jnp.sum(disp_ref[...], axis=0)
        dsum = jnp.sum(dens_ref[...], axis=0)
        val = (E * LBW) * jnp.sum(cnt * dsum) / (nb * nb)
        aux_out[...] = val.reshape(1, 1)


def kernel(multimodal_feat, query_feat, ln_g, ln_b, qe_w1, qe_b1, qe_w2, qe_b2,
           fg_w, fg_b, wg1, wg2, pw1, pb1, pw2, pb2, pw3, pb3, ce_const, ce_wg,
           ce_fc_w, ce_fc_b, cp_w, cp_b, sw1, sb1, sw2, sb2, sw3, sb3, op_w, op_b):
    f32 = jnp.float32
    nb = multimodal_feat.shape[0]
    bta = 1024
    btb = 1024
    na = nb // bta
    ntb = nb // btb

    r2 = lambda v: v.reshape(1, -1)
    fullspec = lambda a, ng: pl.BlockSpec(
        a.shape, functools.partial(lambda nd, *_: (0,) * nd, a.ndim))

    ins_a = [
        multimodal_feat, query_feat, r2(ln_g), r2(ln_b),
        qe_w1, r2(qe_b1), qe_w2, r2(qe_b2),
        fg_w[:H], fg_w[H:], r2(fg_b), wg1, wg2,
        ce_wg, ce_fc_w, r2(ce_fc_b), r2(ce_const), cp_w, r2(cp_b),
    ]
    specs_a = [
        pl.BlockSpec((bta, H), lambda t: (t, 0)),
        pl.BlockSpec((bta, H), lambda t: (t, 0)),
    ] + [fullspec(a, 1) for a in ins_a[2:]]
    out_shape_a = [
        jax.ShapeDtypeStruct((nb, H), f32),
        jax.ShapeDtypeStruct((nb, E), f32),
        jax.ShapeDtypeStruct((nb, P2), f32),
        jax.ShapeDtypeStruct((na, 1, E), f32),
        jax.ShapeDtypeStruct((na, 1, E), f32),
    ]
    out_specs_a = [
        pl.BlockSpec((bta, H), lambda t: (t, 0)),
        pl.BlockSpec((bta, E), lambda t: (t, 0)),
        pl.BlockSpec((bta, P2), lambda t: (t, 0)),
        pl.BlockSpec((1, 1, E), lambda t: (t, 0, 0)),
        pl.BlockSpec((1, 1, E), lambda t: (t, 0, 0)),
    ]
    xn, wd, part, disp, dens = pl.pallas_call(
        _router_body, grid=(na,), in_specs=specs_a,
        out_specs=out_specs_a, out_shape=out_shape_a,
    )(*ins_a)

    ins_s = [
        xn,
        sw1[0], r2(sb1[0]), sw2[0], r2(sb2[0]), sw3[0], r2(sb3[0]),
        sw1[1], r2(sb1[1]), sw2[1], r2(sb2[1]), sw3[1], r2(sb3[1]),
    ]
    specs_s = [pl.BlockSpec((btb, H), lambda t: (t, 0))] + \
        [fullspec(a, 1) for a in ins_s[1:]]
    s0, s1 = pl.pallas_call(
        _shared_body, grid=(ntb,), in_specs=specs_s,
        out_specs=[pl.BlockSpec((btb, P2), lambda t: (t, 0))] * 2,
        out_shape=[jax.ShapeDtypeStruct((nb, P2), f32)] * 2,
    )(*ins_s)

    ins_r = [
        xn, wd, part, s0, s1,
        pw1, pb1.reshape(NREG, 1, 2 * H), pw2, pb2.reshape(NREG, 1, H),
        pw3, pb3.reshape(NREG, 1, P2),
        op_w[0:P2], op_w[P2:2 * P2], op_w[2 * P2:], r2(op_b),
        disp, dens,
    ]
    specs_r = [
        pl.BlockSpec((btb, H), lambda t, e: (t, 0)),
        pl.BlockSpec((btb, E), lambda t, e: (t, 0)),
        pl.BlockSpec((btb, P2), lambda t, e: (t, 0)),
        pl.BlockSpec((btb, P2), lambda t, e: (t, 0)),
        pl.BlockSpec((btb, P2), lambda t, e: (t, 0)),
        pl.BlockSpec((1, H, 2 * H), lambda t, e: (e, 0, 0)),
        pl.BlockSpec((1, 1, 2 * H), lambda t, e: (e, 0, 0)),
        pl.BlockSpec((1, 2 * H, H), lambda t, e: (e, 0, 0)),
        pl.BlockSpec((1, 1, H), lambda t, e: (e, 0, 0)),
        pl.BlockSpec((1, H, P2), lambda t, e: (e, 0, 0)),
        pl.BlockSpec((1, 1, P2), lambda t, e: (e, 0, 0)),
        pl.BlockSpec((P2, P2), lambda t, e: (0, 0)),
        pl.BlockSpec((P2, P2), lambda t, e: (0, 0)),
        pl.BlockSpec((P2, P2), lambda t, e: (0, 0)),
        pl.BlockSpec((1, P2), lambda t, e: (0, 0)),
        pl.BlockSpec((na, 1, E), lambda t, e: (0, 0, 0)),
        pl.BlockSpec((na, 1, E), lambda t, e: (0, 0, 0)),
    ]
    gauss, aux = pl.pallas_call(
        functools.partial(_routed_body, ntb, float(nb)),
        grid=(ntb, NREG),
        in_specs=specs_r,
        out_specs=[
            pl.BlockSpec((btb, P2), lambda t, e: (t, 0)),
            pl.BlockSpec((1, 1), lambda t, e: (0, 0)),
        ],
        out_shape=[
            jax.ShapeDtypeStruct((nb, P2), f32),
            jax.ShapeDtypeStruct((1, 1), f32),
        ],
        scratch_shapes=[pltpu.VMEM((btb, P2), f32)],
    )(*ins_r)

    return gauss.reshape(-1, 2), aux[0, 0]
